# baseline (speedup 1.0000x reference)
import functools

import jax
import jax.numpy as jnp
from jax import lax
from jax.experimental import pallas as pl
from jax.experimental.pallas import tpu as pltpu
from jax.experimental.pallas import tpu_sc as plsc

B = 1024
L = 200
DIM = 64
N = B * L
NPAD = N + 8
PERT = N // 16
IDS_R = 1600
NBUK = 1954
SENT = 1023
CAPW = 33424
IDOF = 16384
STCAP = 2048
TRASH = N


def _body(ids_ref, tabt_ref, tail_ref, out_ref, pairs_sh, meta_sh,
          idx_v, big_v, counts_v, prefix_v, meta_v,
          slab_v, outbuf, sposbuf, sem2):
    cid = lax.axis_index("c")
    wl = lax.axis_index("s")
    lane = lax.iota(jnp.int32, 16)
    prev_perm = jnp.maximum(lane - 1, 0)
    next_perm = jnp.minimum(lane + 1, 15)

    lo = jnp.where(cid == 0, 0, 977)
    hi = jnp.where(cid == 0, 977, NBUK)
    nb = hi - lo

    for i in range(1024 // 16):
        counts_v[pl.ds(i * 16, 16)] = jnp.zeros((16,), jnp.int32)

    def scan_pass(place):
        def round_(r, _):
            pltpu.sync_copy(
                ids_ref.at[pl.ds(wl * PERT + r * IDS_R, IDS_R)], idx_v)

            def group(g, _):
                ids16 = idx_v[pl.ds(g * 16, 16)]
                vb = lax.shift_right_logical(ids16, 9)
                vb = jnp.where(ids16 >= 999872, NBUK - 1, vb)
                mine = (vb >= lo) & (vb < hi)
                vb = jnp.where(mine, vb - lo, SENT)
                skey, slan = plsc.sort_key_val(vb, lane)
                sid = plsc.load_gather(idx_v, [g * 16 + slan])
                prevk = jnp.take(skey, prev_perm)
                nextk = jnp.take(skey, next_perm)
                isfirst = (skey != prevk) | (lane == 0)
                islast = (skey != nextk) | (lane == 15)
                segst = plsc.cummax(jnp.where(isfirst, lane, 0))
                rank = lane - segst
                used = plsc.load_gather(counts_v, [skey])
                plsc.store_scatter(counts_v, [skey], used + rank + 1,
                                   mask=islast)
                if place:
                    base = plsc.load_gather(prefix_v, [skey])
                    slot = base + (used + rank) * 2
                    keep = skey != SENT
                    spos = wl * PERT + r * IDS_R + g * 16 + slan
                    plsc.store_scatter(big_v, [slot], sid, mask=keep)
                    plsc.store_scatter(big_v, [slot + 1], spos, mask=keep)
                return _
            lax.fori_loop(0, IDS_R // 16, group, 0)
            return _
        lax.fori_loop(0, 8, round_, 0)

    scan_pass(False)

    carry = jnp.int32(0)
    for i in range(1024 // 16):
        cnt = counts_v[pl.ds(i * 16, 16)]
        pc = ((cnt + 3) & ~3) * 2
        incl = plsc.cumsum(pc)
        pref = incl - pc + carry
        prefix_v[pl.ds(i * 16, 16)] = pref
        carry = carry + jnp.sum(pc)
        meta_v[pl.ds(i * 16, 16)] = lax.shift_left(pref, 16) | cnt
        counts_v[pl.ds(i * 16, 16)] = jnp.zeros((16,), jnp.int32)

    scan_pass(True)

    pltpu.sync_copy(big_v.at[pl.ds(0, CAPW)],
                    pairs_sh.at[pl.ds(wl * CAPW, CAPW)])
    pltpu.sync_copy(meta_v, meta_sh.at[pl.ds(wl * 1024, 1024)])
    plsc.subcore_barrier()

    pltpu.sync_copy(meta_sh, big_v.at[pl.ds(0, 16384)])
    trash16 = jnp.full((16,), TRASH, jnp.int32)

    def do_bucket(k):
        kg = lo + k
        v0 = jnp.where(kg == NBUK - 1, 999488, kg * 512)

        @pl.when(kg < NBUK - 1)
        def _():
            pltpu.sync_copy(
                tabt_ref.at[:, pl.ds(jnp.minimum(kg, NBUK - 2) * 512, 512)],
                slab_v)

        @pl.when(kg == NBUK - 1)
        def _():
            pltpu.sync_copy(tail_ref, slab_v.at[:, pl.ds(384, 128)])

        col = plsc.load_gather(big_v, [lane * 1024 + k])

        def flush(fill):
            nch = (fill + 127) >> 7

            def chunk(c, _):
                fv = jnp.zeros((16,), jnp.int32)
                for v in range(4):
                    gbase = c * 128 + v * 32
                    ids16 = plsc.load_gather(big_v, [IDOF + gbase + lane * 2])
                    pos16 = plsc.load_gather(
                        big_v, [IDOF + gbase + lane * 2 + 1])
                    vcol = ids16 - v0
                    vcol = jnp.where((vcol >= 0) & (vcol < 512), vcol, 0)
                    valid = (gbase + lane * 2) < fill
                    pos16 = jnp.where(valid, pos16, TRASH)
                    sposbuf[0, pl.ds(v * 16, 16)] = pos16
                    rowv = v * 16 + lane
                    for f in range(DIM):
                        vals = plsc.load_gather(slab_v, [fv, vcol])
                        plsc.store_scatter(outbuf, [rowv, fv], vals)
                        fv = fv + 1
                    fv = fv - DIM
                pltpu.async_copy(
                    outbuf, out_ref.at[sposbuf.at[0]], sem2).wait()
                return _
            lax.fori_loop(0, nch, chunk, 0)

        def src(s, fill):
            m = jnp.max(jnp.where(lane == s, col, 0))
            start = pl.multiple_of(lax.shift_right_logical(m, 16), 8)
            cnt = m & 0xFFFF
            ncc = (cnt + 31) >> 5

            def cell_chunk(c, fl):
                def spill(f):
                    flush(f)
                    return jnp.int32(0)
                fl = lax.cond(fl + 64 > STCAP, spill, lambda f: f, fl)
                fl = pl.multiple_of(fl, 8)
                pltpu.sync_copy(
                    pairs_sh.at[pl.ds(s * CAPW + start + c * 64, 64)],
                    big_v.at[pl.ds(IDOF + fl, 64)])
                take = jnp.minimum(cnt - c * 32, jnp.int32(32))
                for i in range(2):
                    tmask = (i * 16 + lane) >= take
                    plsc.store_scatter(
                        big_v, [IDOF + fl + (i * 16 + lane) * 2 + 1],
                        trash16, mask=tmask)
                return (fl + take * 2 + 7) & ~7

            return lax.fori_loop(0, ncc, cell_chunk, fill)

        fill = lax.fori_loop(0, 16, src, jnp.int32(0))
        flush(fill)

    def bucket_iter(t, carry2):
        k = t * 16 + wl

        @pl.when(k < nb)
        def _():
            do_bucket(k)
        return carry2

    lax.fori_loop(0, 62, bucket_iter, 0)


@jax.jit
def _sc_gather(ids_flat, tabt, tail128):
    mesh = plsc.VectorSubcoreMesh(core_axis_name="c", subcore_axis_name="s")
    return pl.kernel(
        _body,
        out_type=jax.ShapeDtypeStruct((NPAD, 128), jnp.float32),
        mesh=mesh,
        scratch_types=[
            pltpu.VMEM_SHARED((16 * CAPW,), jnp.int32),
            pltpu.VMEM_SHARED((16 * 1024,), jnp.int32),
            pltpu.VMEM((IDS_R,), jnp.int32),
            pltpu.VMEM((CAPW,), jnp.int32),
            pltpu.VMEM((1024,), jnp.int32),
            pltpu.VMEM((1024,), jnp.int32),
            pltpu.VMEM((1024,), jnp.int32),
            pltpu.VMEM((DIM, 512), jnp.float32),
            pltpu.VMEM((64, 128), jnp.float32),
            pltpu.VMEM((8, 64), jnp.int32),
            pltpu.SemaphoreType.DMA,
        ],
        compiler_params=pltpu.CompilerParams(
            use_tc_tiling_on_sc=True, needs_layout_passes=False),
    )(ids_flat, tabt, tail128)


def kernel(input_ids, attention_mask, table):
    ids_flat = input_ids.T.reshape(N).astype(jnp.int32)
    tabt = table.T
    tail128 = table[999872:1000000, :].T
    out128 = _sc_gather(ids_flat, tabt, tail128)
    sel = out128[:N, :DIM]
    tok = sel.reshape(L, B, DIM).transpose(1, 0, 2)
    return (tok, attention_mask)


# speedup vs baseline: 5.7505x; 5.7505x over previous
import functools

import jax
import jax.numpy as jnp
from jax import lax
from jax.experimental import pallas as pl
from jax.experimental.pallas import tpu as pltpu
from jax.experimental.pallas import tpu_sc as plsc

B = 1024
L = 200
DIM = 64
N = B * L
NW = 32
PER_W = N // NW
CHUNK = 128
NCHUNK = PER_W // CHUNK
NBUF = 5
NT = NCHUNK // NBUF


def _gather_body(ids_ref, table_ref, out_ref, idx_v, bufs, gsem, wsem):
    wid = lax.axis_index("s") * 2 + lax.axis_index("c")
    chunk0 = wid * NCHUNK
    pltpu.sync_copy(ids_ref.at[wid], idx_v)

    def fire_gather(c, b):
        pltpu.async_copy(table_ref.at[idx_v.at[c]], bufs.at[b], gsem.at[b])

    def fire_write(c, b):
        pltpu.async_copy(
            bufs.at[b], out_ref.at[pl.ds((chunk0 + c) * CHUNK, CHUNK)],
            wsem.at[b])

    def wait_gather(b):
        pltpu.make_async_copy(
            table_ref.at[idx_v.at[0]], bufs.at[b], gsem.at[b]).wait()

    def wait_write(b):
        pltpu.make_async_copy(
            bufs.at[b], out_ref.at[pl.ds(0, CHUNK)], wsem.at[b]).wait()

    for b in range(NBUF):
        fire_gather(b, b)

    def turn(t, _):
        for b in range(NBUF):
            c = t * NBUF + b
            wait_gather(b)
            fire_write(c, b)
            wait_write(b)
            fire_gather(c + NBUF, b)
        return _

    lax.fori_loop(0, NT - 1, turn, 0)

    for b in range(NBUF):
        c = (NT - 1) * NBUF + b
        wait_gather(b)
        fire_write(c, b)
    for b in range(NBUF):
        wait_write(b)


@jax.jit
def _sc_gather(ids3d, table):
    mesh = plsc.VectorSubcoreMesh(core_axis_name="c", subcore_axis_name="s")
    return pl.kernel(
        _gather_body,
        out_type=jax.ShapeDtypeStruct((N, DIM), jnp.float32),
        mesh=mesh,
        scratch_types=[
            pltpu.VMEM((NCHUNK, CHUNK), jnp.int32),
            pltpu.VMEM((NBUF, CHUNK, DIM), jnp.float32),
            pltpu.SemaphoreType.DMA((NBUF,)),
            pltpu.SemaphoreType.DMA((NBUF,)),
        ],
        compiler_params=pltpu.CompilerParams(use_tc_tiling_on_sc=False),
    )(ids3d, table)


def kernel(input_ids, attention_mask, table):
    ids3d = input_ids.reshape(NW, NCHUNK, CHUNK).astype(jnp.int32)
    flat = _sc_gather(ids3d, table)
    return (flat.reshape(B, L, DIM), attention_mask)


# speedup vs baseline: 6.3842x; 1.1102x over previous
import functools

import jax
import jax.numpy as jnp
from jax import lax
from jax.experimental import pallas as pl
from jax.experimental.pallas import tpu as pltpu
from jax.experimental.pallas import tpu_sc as plsc

B = 1024
L = 200
DIM = 64
N = B * L
NW = 32
PER_W = N // NW
CHUNK = 128
NCHUNK = PER_W // CHUNK
NBUF = 5
NT = NCHUNK // NBUF


def _gather_body(ids_ref, table_ref, out_ref, idx_v, bufs, gsem, wsem):
    wid = lax.axis_index("s") * 2 + lax.axis_index("c")
    chunk0 = wid * NCHUNK
    pltpu.sync_copy(ids_ref.at[wid], idx_v)

    def fire_gather(c, b):
        pltpu.async_copy(table_ref.at[idx_v.at[c]], bufs.at[b], gsem.at[b])

    def fire_write(c, b):
        pltpu.async_copy(
            bufs.at[b],
            out_ref.at[pl.ds((chunk0 + c) * CHUNK, CHUNK), pl.ds(0, DIM)],
            wsem.at[b])

    def wait_gather(b):
        pltpu.make_async_copy(
            table_ref.at[idx_v.at[0]], bufs.at[b], gsem.at[b]).wait()

    def wait_write(b):
        pltpu.make_async_copy(
            bufs.at[b], out_ref.at[pl.ds(0, CHUNK), pl.ds(0, DIM)],
            wsem.at[b]).wait()

    for b in range(NBUF):
        fire_gather(b, b)

    def turn(t, _):
        for b in range(NBUF):
            c = t * NBUF + b
            wait_gather(b)
            fire_write(c, b)
            wait_write(b)
            fire_gather(c + NBUF, b)
        return _

    lax.fori_loop(0, NT - 1, turn, 0)

    for b in range(NBUF):
        c = (NT - 1) * NBUF + b
        wait_gather(b)
        fire_write(c, b)
    for b in range(NBUF):
        wait_write(b)


@jax.jit
def _sc_gather(ids3d, table):
    mesh = plsc.VectorSubcoreMesh(core_axis_name="c", subcore_axis_name="s")
    return pl.kernel(
        _gather_body,
        out_type=jax.ShapeDtypeStruct((N, 128), jnp.float32),
        mesh=mesh,
        scratch_types=[
            pltpu.VMEM((NCHUNK, CHUNK), jnp.int32),
            pltpu.VMEM((NBUF, CHUNK, DIM), jnp.float32),
            pltpu.SemaphoreType.DMA((NBUF,)),
            pltpu.SemaphoreType.DMA((NBUF,)),
        ],
        compiler_params=pltpu.CompilerParams(use_tc_tiling_on_sc=False),
    )(ids3d, table)


def kernel(input_ids, attention_mask, table):
    ids3d = input_ids.reshape(NW, NCHUNK, CHUNK).astype(jnp.int32)
    out128 = _sc_gather(ids3d, table)
    return (out128[:, :DIM].reshape(B, L, DIM), attention_mask)
